# baseline (speedup 1.0000x reference)
import functools
import jax
import jax.numpy as jnp
from jax import lax
from jax.experimental import pallas as pl
from jax.experimental.pallas import tpu as pltpu
from jax.experimental.pallas import tpu_sc as plsc

_NB = 256
_F = 2
_W = 36
_NTAP = 2 * _W + 1
_OFF = _W
_ROWS = 640
_SLAB = 112
_P = 224 * 224
_NCH = 12
_POS = _P // 16
_APP = _P // 32
_SIG_F = 0.01 * 255.0 * _F


def _sc_body(x_hbm, out_hbm, xbuf, cntp, shcnt, slab, tmp0, tmp1, gtab,
             histb, shhist, cdfb, shcdf, ctgtb, lhalf, shlut, lutall,
             abig, obig, dsem, asem, msem, osem):
    core = lax.axis_index("c")
    t = lax.axis_index("s")
    iota16 = lax.broadcasted_iota(jnp.int32, (16,), 0)
    zero16 = jnp.zeros((16,), jnp.float32)
    ones16 = jnp.ones((16,), jnp.float32)
    pb = core * (_P // 2) + t * _APP

    xh = [pltpu.async_copy(x_hbm.at[pl.ds(r * _P + t * _POS, _POS)],
                           xbuf.at[pl.ds(r * _POS, _POS)], dsem)
          for r in range(_NCH)]
    ah = [pltpu.async_copy(x_hbm.at[pl.ds(c * _P + pb, _APP)],
                           abig.at[pl.ds(c * _APP, _APP)], asem)
          for c in range(6)]

    def _zbody(r, _):
        cntp[pl.ds(r * 16, 16)] = zero16
        return 0
    lax.fori_loop(0, _ROWS, _zbody, 0, unroll=8)

    def _gbody(r, _):
        mf = jnp.full((16,), r, jnp.int32).astype(jnp.float32) - float(_W)
        gtab[pl.ds(r * 16, 16)] = jnp.exp(mf * mf * (-0.5 / (_SIG_F * _SIG_F)))
        return 0
    lax.fori_loop(0, _NTAP, _gbody, 0)
    for h in xh:
        h.wait()

    row_base = jnp.minimum(iota16, _NCH - 1) * _POS

    def _sbody(p, idxg):
        xv = plsc.load_gather(xbuf, [idxg])
        ki = (xv * float(255 * _F) + (_OFF + 0.5)).astype(jnp.int32)
        ki = jnp.clip(ki, 0, _ROWS - 1)
        plsc.addupdate_scatter(cntp, [ki * 16 + iota16], ones16)
        return idxg + 1
    lax.fori_loop(0, _POS, _sbody, row_base, unroll=8)

    pltpu.sync_copy(cntp, shcnt.at[t])
    plsc.subcore_barrier()

    r0w = t * (16 * _F * 16)
    tmps = [tmp0, tmp1]
    hs = pltpu.async_copy(shcnt.at[0, pl.ds(r0w, _SLAB * 16)], slab, msem)
    handles = [None] * 16
    handles[1] = pltpu.async_copy(
        shcnt.at[1, pl.ds(r0w, _SLAB * 16)], tmps[1], msem)
    hs.wait()
    for tab in range(1, 16):
        if tab + 1 < 16:
            handles[tab + 1] = pltpu.async_copy(
                shcnt.at[tab + 1, pl.ds(r0w, _SLAB * 16)],
                tmps[(tab + 1) & 1], msem)
        handles[tab].wait()
        buf = tmps[tab & 1]

        def _mbody(r, _, buf=buf):
            sl = pl.ds(r * 16, 16)
            slab[sl] = slab[sl] + buf[sl]
            return 0
        lax.fori_loop(0, 8, _mbody, 0, unroll=8)

    for i in range(16):
        def _cbody(r, acc, i=i):
            return acc + gtab[pl.ds(r * 16, 16)] * slab[pl.ds((_F * i + r) * 16, 16)]
        histb[pl.ds(i * 16, 16)] = lax.fori_loop(0, 9, _cbody, zero16,
                                                 unroll=4)
    pltpu.sync_copy(histb, shhist.at[pl.ds(t * _NB, _NB)])
    plsc.subcore_barrier()

    @pl.when(t == 0)
    def _cdf():
        pltpu.sync_copy(shhist, cdfb)

        def _abody(j, acc):
            sl = pl.ds(j * 16, 16)
            acc = acc + cdfb[sl]
            cdfb[sl] = acc
            return acc
        total = lax.fori_loop(0, _NB, _abody, zero16)
        s = 1.0 / (total + 1e-6)

        def _nbody(j, _):
            sl = pl.ds(j * 16, 16)
            cdfb[sl] = cdfb[sl] * s
            return 0
        lax.fori_loop(0, _NB, _nbody, 0, unroll=4)
        pltpu.sync_copy(cdfb, shcdf)

    plsc.subcore_barrier()

    @pl.when(t < 12)
    def _lut():
        c = t % 6
        half = t // 6
        pltpu.sync_copy(shcdf, cdfb)
        ctile = jnp.full((16,), c, jnp.int32)
        ttile = ctile + 6
        for kb in range(16):
            kidx = iota16 + kb * 16
            ctgtb[pl.ds(kb * 16, 16)] = plsc.load_gather(
                cdfb, [kidx * 16 + ttile])
        for i in range(8):
            kidx = iota16 + half * 128 + i * 16
            v = plsc.load_gather(cdfb, [kidx * 16 + ctile])
            v = jnp.clip(v, 0.0, 1.0)
            pos = jnp.zeros((16,), jnp.int32)
            for step in (128, 64, 32, 16, 8, 4, 2, 1):
                cand = pos + step
                cval = plsc.load_gather(ctgtb, [cand - 1])
                pos = jnp.where(cval <= v, cand, pos)
            idx = jnp.clip(pos, 1, _NB - 1)
            c0 = plsc.load_gather(ctgtb, [idx - 1])
            c1 = plsc.load_gather(ctgtb, [idx])
            tt = (v - c0) / (c1 - c0 + 1e-6)
            lutv = (idx.astype(jnp.float32) - 1.0 + tt) * (1.0 / 255.0)
            lhalf[pl.ds(i * 16, 16)] = jnp.clip(lutv, 0.0, 1.0)
        pltpu.sync_copy(lhalf, shlut.at[pl.ds(c * _NB + half * 128, 128)])

    plsc.subcore_barrier()

    pltpu.sync_copy(shlut, lutall)
    for h in ah:
        h.wait()
    oh = []
    for c in range(6):
        def _pbody(i, _, c=c):
            sl = pl.ds(c * _APP + i * 16, 16)
            v = abig[sl]
            xi = jnp.clip((v * 255.0).astype(jnp.int32), 0, _NB - 1)
            y = plsc.load_gather(lutall, [xi + c * _NB])
            obig[sl] = jnp.clip(y, 0.0, 1.0)
            return 0
        lax.fori_loop(0, 8, _pbody, 0, unroll=4)
        oh.append(pltpu.async_copy(obig.at[pl.ds(c * _APP, _APP)],
                                   out_hbm.at[pl.ds(c * _P + pb, _APP)], osem))
    for h in oh:
        h.wait()


def kernel(source, target):
    N, C, H, W = source.shape
    NC = N * C
    X = jnp.concatenate(
        [source.reshape(NC * _P), target.reshape(NC * _P)], axis=0)
    mesh = plsc.VectorSubcoreMesh(
        core_axis_name="c", subcore_axis_name="s",
        num_cores=2, num_subcores=16)
    fn = functools.partial(
        pl.kernel,
        out_type=jax.ShapeDtypeStruct((NC * _P,), jnp.float32),
        mesh=mesh,
        compiler_params=pltpu.CompilerParams(needs_layout_passes=False),
        scratch_types=[
            pltpu.VMEM((_NCH * _POS,), jnp.float32),
            pltpu.VMEM((_ROWS * 16,), jnp.float32),
            pltpu.VMEM_SHARED((16, _ROWS * 16), jnp.float32),
            pltpu.VMEM((_SLAB * 16,), jnp.float32),
            pltpu.VMEM((_SLAB * 16,), jnp.float32),
            pltpu.VMEM((_SLAB * 16,), jnp.float32),
            pltpu.VMEM((_NTAP * 16,), jnp.float32),
            pltpu.VMEM((_NB,), jnp.float32),
            pltpu.VMEM_SHARED((_NB * 16,), jnp.float32),
            pltpu.VMEM((_NB * 16,), jnp.float32),
            pltpu.VMEM_SHARED((_NB * 16,), jnp.float32),
            pltpu.VMEM((_NB,), jnp.float32),
            pltpu.VMEM((128,), jnp.float32),
            pltpu.VMEM_SHARED((6 * _NB,), jnp.float32),
            pltpu.VMEM((6 * _NB,), jnp.float32),
            pltpu.VMEM((6 * _APP,), jnp.float32),
            pltpu.VMEM((6 * _APP,), jnp.float32),
            pltpu.SemaphoreType.DMA,
            pltpu.SemaphoreType.DMA,
            pltpu.SemaphoreType.DMA,
            pltpu.SemaphoreType.DMA,
        ],
    )(_sc_body)
    out = fn(X)
    return jnp.clip(out.reshape(N, C, H, W), 0.0, 1.0)


# speedup vs baseline: 1.1511x; 1.1511x over previous
import functools
import jax
import jax.numpy as jnp
from jax import lax
from jax.experimental import pallas as pl
from jax.experimental.pallas import tpu as pltpu
from jax.experimental.pallas import tpu_sc as plsc

_NB = 256
_F = 2
_W = 36
_NTAP = 2 * _W + 1
_OFF = _W
_ROWS = 640
_SLAB = 112
_P = 224 * 224
_NCH = 12
_POS = _P // 16
_APP = _P // 32
_SIG_F = 0.01 * 255.0 * _F


def _sc_body(x_hbm, out_hbm, xbuf, cntp, shcnt, slab, tmp0, tmp1, gtab,
             histb, shhist, cdfb, shcdf, ctgtb, lhalf, shlut, lutall,
             abig, obig, dsem, asem, msem, osem):
    core = lax.axis_index("c")
    t = lax.axis_index("s")
    iota16 = lax.broadcasted_iota(jnp.int32, (16,), 0)
    zero16 = jnp.zeros((16,), jnp.float32)
    ones16 = jnp.ones((16,), jnp.float32)
    pb = core * (_P // 2) + t * _APP

    xh = [pltpu.async_copy(x_hbm.at[pl.ds(r * _P + t * _POS, _POS)],
                           xbuf.at[pl.ds(r * _POS, _POS)], dsem)
          for r in range(_NCH)]
    ah = [pltpu.async_copy(x_hbm.at[pl.ds(c * _P + pb, _APP)],
                           abig.at[pl.ds(c * _APP, _APP)], asem)
          for c in range(6)]

    def _zbody(r, _):
        cntp[pl.ds(r * 16, 16)] = zero16
        return 0
    lax.fori_loop(0, _ROWS, _zbody, 0, unroll=8)

    def _gbody(r, _):
        mf = jnp.full((16,), r, jnp.int32).astype(jnp.float32) - float(_W)
        gtab[pl.ds(r * 16, 16)] = jnp.exp(mf * mf * (-0.5 / (_SIG_F * _SIG_F)))
        return 0
    lax.fori_loop(0, _NTAP, _gbody, 0)
    for h in xh:
        h.wait()

    row_base = jnp.minimum(iota16, _NCH - 1) * _POS
    row_lim = row_base + _POS

    def _sbody(p, idxg):
        xv = plsc.load_gather(xbuf, [idxg])
        ki = (xv * float(255 * _F) + (_OFF + 0.5)).astype(jnp.int32)
        ki = jnp.clip(ki, 0, _ROWS - 1)
        plsc.addupdate_scatter(cntp, [ki * 16 + iota16], ones16)
        nxt = idxg + 1
        return jnp.where(nxt == row_lim, row_base, nxt)
    lax.fori_loop(0, _POS, _sbody, row_base + iota16 * 131, unroll=8)

    pltpu.sync_copy(cntp, shcnt.at[t])
    plsc.subcore_barrier()

    r0w = t * (16 * _F * 16)
    tmps = [tmp0, tmp1]
    hs = pltpu.async_copy(shcnt.at[0, pl.ds(r0w, _SLAB * 16)], slab, msem)
    handles = [None] * 16
    handles[1] = pltpu.async_copy(
        shcnt.at[1, pl.ds(r0w, _SLAB * 16)], tmps[1], msem)
    hs.wait()
    for tab in range(1, 16):
        if tab + 1 < 16:
            handles[tab + 1] = pltpu.async_copy(
                shcnt.at[tab + 1, pl.ds(r0w, _SLAB * 16)],
                tmps[(tab + 1) & 1], msem)
        handles[tab].wait()
        buf = tmps[tab & 1]

        def _mbody(r, _, buf=buf):
            sl = pl.ds(r * 16, 16)
            slab[sl] = slab[sl] + buf[sl]
            return 0
        lax.fori_loop(0, _SLAB, _mbody, 0, unroll=8)

    for i in range(16):
        def _cbody(r, acc, i=i):
            return acc + gtab[pl.ds(r * 16, 16)] * slab[pl.ds((_F * i + r) * 16, 16)]
        histb[pl.ds(i * 16, 16)] = lax.fori_loop(0, _NTAP, _cbody, zero16,
                                                 unroll=4)
    pltpu.sync_copy(histb, shhist.at[pl.ds(t * _NB, _NB)])
    plsc.subcore_barrier()

    @pl.when(t == 0)
    def _cdf():
        pltpu.sync_copy(shhist, cdfb)

        def _abody(j, acc):
            sl = pl.ds(j * 16, 16)
            acc = acc + cdfb[sl]
            cdfb[sl] = acc
            return acc
        total = lax.fori_loop(0, _NB, _abody, zero16)
        s = 1.0 / (total + 1e-6)

        def _nbody(j, _):
            sl = pl.ds(j * 16, 16)
            cdfb[sl] = cdfb[sl] * s
            return 0
        lax.fori_loop(0, _NB, _nbody, 0, unroll=4)
        pltpu.sync_copy(cdfb, shcdf)

    plsc.subcore_barrier()

    @pl.when(t < 12)
    def _lut():
        c = t % 6
        half = t // 6
        pltpu.sync_copy(shcdf, cdfb)
        ctile = jnp.full((16,), c, jnp.int32)
        ttile = ctile + 6
        for kb in range(16):
            kidx = iota16 + kb * 16
            ctgtb[pl.ds(kb * 16, 16)] = plsc.load_gather(
                cdfb, [kidx * 16 + ttile])
        for i in range(8):
            kidx = iota16 + half * 128 + i * 16
            v = plsc.load_gather(cdfb, [kidx * 16 + ctile])
            v = jnp.clip(v, 0.0, 1.0)
            pos = jnp.zeros((16,), jnp.int32)
            for step in (128, 64, 32, 16, 8, 4, 2, 1):
                cand = pos + step
                cval = plsc.load_gather(ctgtb, [cand - 1])
                pos = jnp.where(cval <= v, cand, pos)
            idx = jnp.clip(pos, 1, _NB - 1)
            c0 = plsc.load_gather(ctgtb, [idx - 1])
            c1 = plsc.load_gather(ctgtb, [idx])
            tt = (v - c0) / (c1 - c0 + 1e-6)
            lutv = (idx.astype(jnp.float32) - 1.0 + tt) * (1.0 / 255.0)
            lhalf[pl.ds(i * 16, 16)] = jnp.clip(lutv, 0.0, 1.0)
        pltpu.sync_copy(lhalf, shlut.at[pl.ds(c * _NB + half * 128, 128)])

    plsc.subcore_barrier()

    pltpu.sync_copy(shlut, lutall)
    for h in ah:
        h.wait()
    oh = []
    for c in range(6):
        def _pbody(i, _, c=c):
            sl = pl.ds(c * _APP + i * 16, 16)
            v = abig[sl]
            xi = jnp.clip((v * 255.0).astype(jnp.int32), 0, _NB - 1)
            y = plsc.load_gather(lutall, [xi + c * _NB])
            obig[sl] = jnp.clip(y, 0.0, 1.0)
            return 0
        lax.fori_loop(0, _APP // 16, _pbody, 0, unroll=4)
        oh.append(pltpu.async_copy(obig.at[pl.ds(c * _APP, _APP)],
                                   out_hbm.at[pl.ds(c * _P + pb, _APP)], osem))
    for h in oh:
        h.wait()


def kernel(source, target):
    N, C, H, W = source.shape
    NC = N * C
    X = jnp.concatenate(
        [source.reshape(NC * _P), target.reshape(NC * _P)], axis=0)
    mesh = plsc.VectorSubcoreMesh(
        core_axis_name="c", subcore_axis_name="s",
        num_cores=2, num_subcores=16)
    fn = functools.partial(
        pl.kernel,
        out_type=jax.ShapeDtypeStruct((NC * _P,), jnp.float32),
        mesh=mesh,
        compiler_params=pltpu.CompilerParams(needs_layout_passes=False),
        scratch_types=[
            pltpu.VMEM((_NCH * _POS,), jnp.float32),
            pltpu.VMEM((_ROWS * 16,), jnp.float32),
            pltpu.VMEM_SHARED((16, _ROWS * 16), jnp.float32),
            pltpu.VMEM((_SLAB * 16,), jnp.float32),
            pltpu.VMEM((_SLAB * 16,), jnp.float32),
            pltpu.VMEM((_SLAB * 16,), jnp.float32),
            pltpu.VMEM((_NTAP * 16,), jnp.float32),
            pltpu.VMEM((_NB,), jnp.float32),
            pltpu.VMEM_SHARED((_NB * 16,), jnp.float32),
            pltpu.VMEM((_NB * 16,), jnp.float32),
            pltpu.VMEM_SHARED((_NB * 16,), jnp.float32),
            pltpu.VMEM((_NB,), jnp.float32),
            pltpu.VMEM((128,), jnp.float32),
            pltpu.VMEM_SHARED((6 * _NB,), jnp.float32),
            pltpu.VMEM((6 * _NB,), jnp.float32),
            pltpu.VMEM((6 * _APP,), jnp.float32),
            pltpu.VMEM((6 * _APP,), jnp.float32),
            pltpu.SemaphoreType.DMA,
            pltpu.SemaphoreType.DMA,
            pltpu.SemaphoreType.DMA,
            pltpu.SemaphoreType.DMA,
        ],
    )(_sc_body)
    out = fn(X)
    return jnp.clip(out.reshape(N, C, H, W), 0.0, 1.0)


# speedup vs baseline: 1.1719x; 1.0181x over previous
import functools
import jax
import jax.numpy as jnp
from jax import lax
from jax.experimental import pallas as pl
from jax.experimental.pallas import tpu as pltpu
from jax.experimental.pallas import tpu_sc as plsc

_NB = 256
_F = 1
_W = 18
_NTAP = 2 * _W + 1
_OFF = _W
_ROWS = 320
_SLAB = 56
_P = 224 * 224
_NCH = 12
_POS = _P // 16
_APP = _P // 32
_SIG_F = 0.01 * 255.0 * _F


def _sc_body(x_hbm, out_hbm, xbuf, cntp, shcnt, slab, tmp0, tmp1, gtab,
             histb, shhist, cdfb, shcdf, ctgtb, lhalf, shlut, lutall,
             abig, obig, dsem, asem, msem, osem):
    core = lax.axis_index("c")
    t = lax.axis_index("s")
    iota16 = lax.broadcasted_iota(jnp.int32, (16,), 0)
    zero16 = jnp.zeros((16,), jnp.float32)
    ones16 = jnp.ones((16,), jnp.float32)
    pb = core * (_P // 2) + t * _APP

    xh = [pltpu.async_copy(x_hbm.at[pl.ds(r * _P + t * _POS, _POS)],
                           xbuf.at[pl.ds(r * _POS, _POS)], dsem)
          for r in range(_NCH)]
    ah = [pltpu.async_copy(x_hbm.at[pl.ds(c * _P + pb, _APP)],
                           abig.at[pl.ds(c * _APP, _APP)], asem)
          for c in range(6)]

    def _zbody(r, _):
        cntp[pl.ds(r * 16, 16)] = zero16
        return 0
    lax.fori_loop(0, _ROWS, _zbody, 0, unroll=8)

    def _gbody(r, _):
        mf = jnp.full((16,), r, jnp.int32).astype(jnp.float32) - float(_W)
        gtab[pl.ds(r * 16, 16)] = jnp.exp(mf * mf * (-0.5 / (_SIG_F * _SIG_F)))
        return 0
    lax.fori_loop(0, _NTAP, _gbody, 0)
    for h in xh:
        h.wait()

    row_base = jnp.minimum(iota16, _NCH - 1) * _POS
    row_lim = row_base + _POS

    def _sbody(p, idxg):
        xv = plsc.load_gather(xbuf, [idxg])
        ki = (xv * float(255 * _F) + (_OFF + 0.5)).astype(jnp.int32)
        ki = jnp.clip(ki, 0, _ROWS - 1)
        plsc.addupdate_scatter(cntp, [ki * 16 + iota16], ones16)
        nxt = idxg + 1
        return jnp.where(nxt == row_lim, row_base, nxt)
    lax.fori_loop(0, _POS, _sbody, row_base + iota16 * 131, unroll=8)

    pltpu.sync_copy(cntp, shcnt.at[t])
    plsc.subcore_barrier()

    r0w = t * (16 * _F * 16)
    tmps = [tmp0, tmp1]
    hs = pltpu.async_copy(shcnt.at[0, pl.ds(r0w, _SLAB * 16)], slab, msem)
    handles = [None] * 16
    handles[1] = pltpu.async_copy(
        shcnt.at[1, pl.ds(r0w, _SLAB * 16)], tmps[1], msem)
    hs.wait()
    for tab in range(1, 16):
        if tab + 1 < 16:
            handles[tab + 1] = pltpu.async_copy(
                shcnt.at[tab + 1, pl.ds(r0w, _SLAB * 16)],
                tmps[(tab + 1) & 1], msem)
        handles[tab].wait()
        buf = tmps[tab & 1]

        def _mbody(r, _, buf=buf):
            sl = pl.ds(r * 16, 16)
            slab[sl] = slab[sl] + buf[sl]
            return 0
        lax.fori_loop(0, _SLAB, _mbody, 0, unroll=8)

    for i in range(16):
        def _cbody(r, acc, i=i):
            return acc + gtab[pl.ds(r * 16, 16)] * slab[pl.ds((_F * i + r) * 16, 16)]
        histb[pl.ds(i * 16, 16)] = lax.fori_loop(0, _NTAP, _cbody, zero16,
                                                 unroll=4)
    pltpu.sync_copy(histb, shhist.at[pl.ds(t * _NB, _NB)])
    plsc.subcore_barrier()

    @pl.when(t == 0)
    def _cdf():
        pltpu.sync_copy(shhist, cdfb)

        def _abody(j, acc):
            sl = pl.ds(j * 16, 16)
            acc = acc + cdfb[sl]
            cdfb[sl] = acc
            return acc
        total = lax.fori_loop(0, _NB, _abody, zero16)
        s = 1.0 / (total + 1e-6)

        def _nbody(j, _):
            sl = pl.ds(j * 16, 16)
            cdfb[sl] = cdfb[sl] * s
            return 0
        lax.fori_loop(0, _NB, _nbody, 0, unroll=4)
        pltpu.sync_copy(cdfb, shcdf)

    plsc.subcore_barrier()

    @pl.when(t < 12)
    def _lut():
        c = t % 6
        half = t // 6
        pltpu.sync_copy(shcdf, cdfb)
        ctile = jnp.full((16,), c, jnp.int32)
        ttile = ctile + 6
        for kb in range(16):
            kidx = iota16 + kb * 16
            ctgtb[pl.ds(kb * 16, 16)] = plsc.load_gather(
                cdfb, [kidx * 16 + ttile])
        for i in range(8):
            kidx = iota16 + half * 128 + i * 16
            v = plsc.load_gather(cdfb, [kidx * 16 + ctile])
            v = jnp.clip(v, 0.0, 1.0)
            pos = jnp.zeros((16,), jnp.int32)
            for step in (128, 64, 32, 16, 8, 4, 2, 1):
                cand = pos + step
                cval = plsc.load_gather(ctgtb, [cand - 1])
                pos = jnp.where(cval <= v, cand, pos)
            idx = jnp.clip(pos, 1, _NB - 1)
            c0 = plsc.load_gather(ctgtb, [idx - 1])
            c1 = plsc.load_gather(ctgtb, [idx])
            tt = (v - c0) / (c1 - c0 + 1e-6)
            lutv = (idx.astype(jnp.float32) - 1.0 + tt) * (1.0 / 255.0)
            lhalf[pl.ds(i * 16, 16)] = jnp.clip(lutv, 0.0, 1.0)
        pltpu.sync_copy(lhalf, shlut.at[pl.ds(c * _NB + half * 128, 128)])

    plsc.subcore_barrier()

    pltpu.sync_copy(shlut, lutall)
    for h in ah:
        h.wait()
    oh = []
    for c in range(6):
        def _pbody(i, _, c=c):
            sl = pl.ds(c * _APP + i * 16, 16)
            v = abig[sl]
            xi = jnp.clip((v * 255.0).astype(jnp.int32), 0, _NB - 1)
            y = plsc.load_gather(lutall, [xi + c * _NB])
            obig[sl] = jnp.clip(y, 0.0, 1.0)
            return 0
        lax.fori_loop(0, _APP // 16, _pbody, 0, unroll=4)
        oh.append(pltpu.async_copy(obig.at[pl.ds(c * _APP, _APP)],
                                   out_hbm.at[pl.ds(c * _P + pb, _APP)], osem))
    for h in oh:
        h.wait()


def kernel(source, target):
    N, C, H, W = source.shape
    NC = N * C
    X = jnp.concatenate(
        [source.reshape(NC * _P), target.reshape(NC * _P)], axis=0)
    mesh = plsc.VectorSubcoreMesh(
        core_axis_name="c", subcore_axis_name="s",
        num_cores=2, num_subcores=16)
    fn = functools.partial(
        pl.kernel,
        out_type=jax.ShapeDtypeStruct((NC * _P,), jnp.float32),
        mesh=mesh,
        compiler_params=pltpu.CompilerParams(needs_layout_passes=False),
        scratch_types=[
            pltpu.VMEM((_NCH * _POS,), jnp.float32),
            pltpu.VMEM((_ROWS * 16,), jnp.float32),
            pltpu.VMEM_SHARED((16, _ROWS * 16), jnp.float32),
            pltpu.VMEM((_SLAB * 16,), jnp.float32),
            pltpu.VMEM((_SLAB * 16,), jnp.float32),
            pltpu.VMEM((_SLAB * 16,), jnp.float32),
            pltpu.VMEM((_NTAP * 16,), jnp.float32),
            pltpu.VMEM((_NB,), jnp.float32),
            pltpu.VMEM_SHARED((_NB * 16,), jnp.float32),
            pltpu.VMEM((_NB * 16,), jnp.float32),
            pltpu.VMEM_SHARED((_NB * 16,), jnp.float32),
            pltpu.VMEM((_NB,), jnp.float32),
            pltpu.VMEM((128,), jnp.float32),
            pltpu.VMEM_SHARED((6 * _NB,), jnp.float32),
            pltpu.VMEM((6 * _NB,), jnp.float32),
            pltpu.VMEM((6 * _APP,), jnp.float32),
            pltpu.VMEM((6 * _APP,), jnp.float32),
            pltpu.SemaphoreType.DMA,
            pltpu.SemaphoreType.DMA,
            pltpu.SemaphoreType.DMA,
            pltpu.SemaphoreType.DMA,
        ],
    )(_sc_body)
    out = fn(X)
    return jnp.clip(out.reshape(N, C, H, W), 0.0, 1.0)


# speedup vs baseline: 1.7750x; 1.5146x over previous
import functools
import jax
import jax.numpy as jnp
from jax import lax
from jax.experimental import pallas as pl
from jax.experimental.pallas import tpu as pltpu
from jax.experimental.pallas import tpu_sc as plsc

_NB = 256
_F = 1
_W = 18
_NTAP = 2 * _W + 1
_OFF = _W
_ROWS = 320
_SLAB = 56
_P = 224 * 224
_NCH = 12
_POS = _P // 16
_APP = _P // 32
_SIG_F = 0.01 * 255.0 * _F
_XSTR = _POS + 8


def _sc_body(x_hbm, out_hbm, xbuf, cntp, shcnt, slab, tmp0, tmp1, gtab,
             histb, shhist, cdfb, shcdf, ctgtb, lhalf, shlut, lutall,
             abig, obig, dsem, asem, msem, osem):
    core = lax.axis_index("c")
    t = lax.axis_index("s")
    iota16 = lax.broadcasted_iota(jnp.int32, (16,), 0)
    zero16 = jnp.zeros((16,), jnp.float32)
    ones16 = jnp.ones((16,), jnp.float32)
    pb = core * (_P // 2) + t * _APP

    xh = []
    for r in range(_NCH):
        xh.append(pltpu.async_copy(x_hbm.at[pl.ds(r * _P + t * _POS, _POS)],
                                   xbuf.at[pl.ds(r * _XSTR, _POS)], dsem))
        xh.append(pltpu.async_copy(x_hbm.at[pl.ds(r * _P + t * _POS, 8)],
                                   xbuf.at[pl.ds(r * _XSTR + _POS, 8)], dsem))
    ah = [pltpu.async_copy(x_hbm.at[pl.ds(c * _P + pb, _APP)],
                           abig.at[pl.ds(c * _APP, _APP)], asem)
          for c in range(6)]

    def _zbody(r, _):
        cntp[pl.ds(r * 16, 16)] = zero16
        return 0
    lax.fori_loop(0, _ROWS, _zbody, 0, unroll=8)

    def _gbody(r, _):
        mf = jnp.full((16,), r, jnp.int32).astype(jnp.float32) - float(_W)
        gtab[pl.ds(r * 16, 16)] = jnp.exp(mf * mf * (-0.5 / (_SIG_F * _SIG_F)))
        return 0
    lax.fori_loop(0, _NTAP, _gbody, 0)
    for h in xh:
        h.wait()

    row_base = jnp.minimum(iota16, _NCH - 1) * _XSTR
    row_lim = row_base + _POS

    def _sbody(p, idxg):
        xs = [plsc.load_gather(xbuf, [idxg + k]) for k in range(8)]
        kis = [jnp.clip((xv * float(255 * _F) + (_OFF + 0.5))
                        .astype(jnp.int32), 0, _ROWS - 1) * 16 + iota16
               for xv in xs]
        for kv in kis:
            plsc.addupdate_scatter(cntp, [kv], ones16)
        nxt = idxg + 8
        return jnp.where(nxt >= row_lim, nxt - _POS, nxt)
    lax.fori_loop(0, _POS // 8, _sbody, row_base + iota16 * 131, unroll=2)

    pltpu.sync_copy(cntp, shcnt.at[t])
    plsc.subcore_barrier()

    r0w = t * (16 * _F * 16)
    tmps = [tmp0, tmp1]
    hs = pltpu.async_copy(shcnt.at[0, pl.ds(r0w, _SLAB * 16)], slab, msem)
    handles = [None] * 16
    handles[1] = pltpu.async_copy(
        shcnt.at[1, pl.ds(r0w, _SLAB * 16)], tmps[1], msem)
    hs.wait()
    for tab in range(1, 16):
        if tab + 1 < 16:
            handles[tab + 1] = pltpu.async_copy(
                shcnt.at[tab + 1, pl.ds(r0w, _SLAB * 16)],
                tmps[(tab + 1) & 1], msem)
        handles[tab].wait()
        buf = tmps[tab & 1]

        def _mbody(r, _, buf=buf):
            sl = pl.ds(r * 16, 16)
            slab[sl] = slab[sl] + buf[sl]
            return 0
        lax.fori_loop(0, _SLAB, _mbody, 0, unroll=8)

    for i in range(16):
        def _cbody(r, acc, i=i):
            return acc + gtab[pl.ds(r * 16, 16)] * slab[pl.ds((_F * i + r) * 16, 16)]
        histb[pl.ds(i * 16, 16)] = lax.fori_loop(0, _NTAP, _cbody, zero16,
                                                 unroll=4)
    pltpu.sync_copy(histb, shhist.at[pl.ds(t * _NB, _NB)])
    plsc.subcore_barrier()

    @pl.when(t == 0)
    def _cdf():
        pltpu.sync_copy(shhist, cdfb)

        def _abody(j, acc):
            sl = pl.ds(j * 16, 16)
            acc = acc + cdfb[sl]
            cdfb[sl] = acc
            return acc
        total = lax.fori_loop(0, _NB, _abody, zero16)
        s = 1.0 / (total + 1e-6)

        def _nbody(j, _):
            sl = pl.ds(j * 16, 16)
            cdfb[sl] = cdfb[sl] * s
            return 0
        lax.fori_loop(0, _NB, _nbody, 0, unroll=4)
        pltpu.sync_copy(cdfb, shcdf)

    plsc.subcore_barrier()

    @pl.when(t < 12)
    def _lut():
        c = t % 6
        half = t // 6
        pltpu.sync_copy(shcdf, cdfb)
        ctile = jnp.full((16,), c, jnp.int32)
        ttile = ctile + 6
        for kb in range(16):
            kidx = iota16 + kb * 16
            ctgtb[pl.ds(kb * 16, 16)] = plsc.load_gather(
                cdfb, [kidx * 16 + ttile])
        for i in range(8):
            kidx = iota16 + half * 128 + i * 16
            v = plsc.load_gather(cdfb, [kidx * 16 + ctile])
            v = jnp.clip(v, 0.0, 1.0)
            pos = jnp.zeros((16,), jnp.int32)
            for step in (128, 64, 32, 16, 8, 4, 2, 1):
                cand = pos + step
                cval = plsc.load_gather(ctgtb, [cand - 1])
                pos = jnp.where(cval <= v, cand, pos)
            idx = jnp.clip(pos, 1, _NB - 1)
            c0 = plsc.load_gather(ctgtb, [idx - 1])
            c1 = plsc.load_gather(ctgtb, [idx])
            tt = (v - c0) / (c1 - c0 + 1e-6)
            lutv = (idx.astype(jnp.float32) - 1.0 + tt) * (1.0 / 255.0)
            lhalf[pl.ds(i * 16, 16)] = jnp.clip(lutv, 0.0, 1.0)
        pltpu.sync_copy(lhalf, shlut.at[pl.ds(c * _NB + half * 128, 128)])

    plsc.subcore_barrier()

    pltpu.sync_copy(shlut, lutall)
    for h in ah:
        h.wait()
    oh = []
    for c in range(6):
        def _pbody(i, _, c=c):
            sl = pl.ds(c * _APP + i * 16, 16)
            v = abig[sl]
            xi = jnp.clip((v * 255.0).astype(jnp.int32), 0, _NB - 1)
            y = plsc.load_gather(lutall, [xi + c * _NB])
            obig[sl] = jnp.clip(y, 0.0, 1.0)
            return 0
        lax.fori_loop(0, _APP // 16, _pbody, 0, unroll=4)
        oh.append(pltpu.async_copy(obig.at[pl.ds(c * _APP, _APP)],
                                   out_hbm.at[pl.ds(c * _P + pb, _APP)], osem))
    for h in oh:
        h.wait()


def kernel(source, target):
    N, C, H, W = source.shape
    NC = N * C
    X = jnp.concatenate(
        [source.reshape(NC * _P), target.reshape(NC * _P)], axis=0)
    mesh = plsc.VectorSubcoreMesh(
        core_axis_name="c", subcore_axis_name="s",
        num_cores=2, num_subcores=16)
    fn = functools.partial(
        pl.kernel,
        out_type=jax.ShapeDtypeStruct((NC * _P,), jnp.float32),
        mesh=mesh,
        compiler_params=pltpu.CompilerParams(needs_layout_passes=False),
        scratch_types=[
            pltpu.VMEM((_NCH * _XSTR,), jnp.float32),
            pltpu.VMEM((_ROWS * 16,), jnp.float32),
            pltpu.VMEM_SHARED((16, _ROWS * 16), jnp.float32),
            pltpu.VMEM((_SLAB * 16,), jnp.float32),
            pltpu.VMEM((_SLAB * 16,), jnp.float32),
            pltpu.VMEM((_SLAB * 16,), jnp.float32),
            pltpu.VMEM((_NTAP * 16,), jnp.float32),
            pltpu.VMEM((_NB,), jnp.float32),
            pltpu.VMEM_SHARED((_NB * 16,), jnp.float32),
            pltpu.VMEM((_NB * 16,), jnp.float32),
            pltpu.VMEM_SHARED((_NB * 16,), jnp.float32),
            pltpu.VMEM((_NB,), jnp.float32),
            pltpu.VMEM((128,), jnp.float32),
            pltpu.VMEM_SHARED((6 * _NB,), jnp.float32),
            pltpu.VMEM((6 * _NB,), jnp.float32),
            pltpu.VMEM((6 * _APP,), jnp.float32),
            pltpu.VMEM((6 * _APP,), jnp.float32),
            pltpu.SemaphoreType.DMA,
            pltpu.SemaphoreType.DMA,
            pltpu.SemaphoreType.DMA,
            pltpu.SemaphoreType.DMA,
        ],
    )(_sc_body)
    out = fn(X)
    return jnp.clip(out.reshape(N, C, H, W), 0.0, 1.0)


# speedup vs baseline: 2.0938x; 1.1796x over previous
import functools
import jax
import jax.numpy as jnp
from jax import lax
from jax.experimental import pallas as pl
from jax.experimental.pallas import tpu as pltpu
from jax.experimental.pallas import tpu_sc as plsc

_NB = 256
_F = 1
_W = 18
_NTAP = 2 * _W + 1
_OFF = _W
_ROWS = 320
_SLAB = 56
_P = 224 * 224
_NCH = 12
_POS = _P // 16
_APP = _P // 32
_SIG_F = 0.01 * 255.0 * _F
_XSTR = _POS + 8


def _sc_body(xs_hbm, xt_hbm, out_hbm, xbuf, cntp, shcnt, slab, tmpbig, gtab,
             histb, shhist, cdfb, shcdf, ctgtb, lhalf, shlut, lutall,
             abig, obig, dsem, asem, msem, osem):
    core = lax.axis_index("c")
    t = lax.axis_index("s")
    iota16 = lax.broadcasted_iota(jnp.int32, (16,), 0)
    zero16 = jnp.zeros((16,), jnp.float32)
    ones16 = jnp.ones((16,), jnp.float32)
    pb = core * (_P // 2) + t * _APP

    xh = []
    for r in range(_NCH):
        src_ref = xs_hbm if r < 6 else xt_hbm
        off = (r % 6) * _P + t * _POS
        xh.append(pltpu.async_copy(src_ref.at[pl.ds(off, _POS)],
                                   xbuf.at[pl.ds(r * _XSTR, _POS)], dsem))
        xh.append(pltpu.async_copy(src_ref.at[pl.ds(off, 8)],
                                   xbuf.at[pl.ds(r * _XSTR + _POS, 8)], dsem))
    ah = [pltpu.async_copy(xs_hbm.at[pl.ds(c * _P + pb, _APP)],
                           abig.at[pl.ds(c * _APP, _APP)], asem)
          for c in range(6)]

    def _zbody(r, _):
        cntp[pl.ds(r * 16, 16)] = zero16
        return 0
    lax.fori_loop(0, _ROWS, _zbody, 0, unroll=8)

    def _gbody(r, _):
        mf = jnp.full((16,), r, jnp.int32).astype(jnp.float32) - float(_W)
        gtab[pl.ds(r * 16, 16)] = jnp.exp(mf * mf * (-0.5 / (_SIG_F * _SIG_F)))
        return 0
    lax.fori_loop(0, _NTAP, _gbody, 0)
    for h in xh:
        h.wait()

    row_base = jnp.minimum(iota16, _NCH - 1) * _XSTR
    row_lim = row_base + _POS

    def _sbody(p, idxg):
        xs = [plsc.load_gather(xbuf, [idxg + k]) for k in range(8)]
        kis = [jnp.clip((xv * float(255 * _F) + (_OFF + 0.5))
                        .astype(jnp.int32), 0, _ROWS - 1) * 16 + iota16
               for xv in xs]
        for kv in kis:
            plsc.addupdate_scatter(cntp, [kv], ones16)
        nxt = idxg + 8
        return jnp.where(nxt >= row_lim, nxt - _POS, nxt)
    lax.fori_loop(0, _POS // 8, _sbody, row_base + iota16 * 131, unroll=2)

    pltpu.sync_copy(cntp, shcnt.at[t])
    plsc.subcore_barrier()

    r0w = t * (16 * _F * 16)
    mh = [pltpu.async_copy(shcnt.at[0, pl.ds(r0w, _SLAB * 16)], slab, msem)]
    for tab in range(1, 16):
        mh.append(pltpu.async_copy(
            shcnt.at[tab, pl.ds(r0w, _SLAB * 16)],
            tmpbig.at[pl.ds((tab - 1) * _SLAB * 16, _SLAB * 16)], msem))
    for h in mh:
        h.wait()

    def _mbody(r, _):
        sl = pl.ds(r * 16, 16)
        acc = slab[sl]
        accs = [tmpbig[pl.ds(((tab * _SLAB) + r) * 16, 16)]
                for tab in range(15)]
        for a in accs:
            acc = acc + a
        slab[sl] = acc
        return 0
    lax.fori_loop(0, _SLAB, _mbody, 0, unroll=2)

    for i in range(16):
        def _cbody(r, acc, i=i):
            return acc + gtab[pl.ds(r * 16, 16)] * slab[pl.ds((_F * i + r) * 16, 16)]
        histb[pl.ds(i * 16, 16)] = lax.fori_loop(0, _NTAP, _cbody, zero16,
                                                 unroll=4)
    pltpu.sync_copy(histb, shhist.at[pl.ds(t * _NB, _NB)])
    plsc.subcore_barrier()

    @pl.when(t == 0)
    def _cdf():
        pltpu.sync_copy(shhist, cdfb)

        def _abody(j, acc):
            sl = pl.ds(j * 16, 16)
            acc = acc + cdfb[sl]
            cdfb[sl] = acc
            return acc
        total = lax.fori_loop(0, _NB, _abody, zero16)
        s = 1.0 / (total + 1e-6)

        def _nbody(j, _):
            sl = pl.ds(j * 16, 16)
            cdfb[sl] = cdfb[sl] * s
            return 0
        lax.fori_loop(0, _NB, _nbody, 0, unroll=4)
        pltpu.sync_copy(cdfb, shcdf)

    plsc.subcore_barrier()

    @pl.when(t < 12)
    def _lut():
        c = t % 6
        half = t // 6
        pltpu.sync_copy(shcdf, cdfb)
        ctile = jnp.full((16,), c, jnp.int32)
        ttile = ctile + 6
        for kb in range(16):
            kidx = iota16 + kb * 16
            ctgtb[pl.ds(kb * 16, 16)] = plsc.load_gather(
                cdfb, [kidx * 16 + ttile])
        for i in range(8):
            kidx = iota16 + half * 128 + i * 16
            v = plsc.load_gather(cdfb, [kidx * 16 + ctile])
            v = jnp.clip(v, 0.0, 1.0)
            pos = jnp.zeros((16,), jnp.int32)
            for step in (128, 64, 32, 16, 8, 4, 2, 1):
                cand = pos + step
                cval = plsc.load_gather(ctgtb, [cand - 1])
                pos = jnp.where(cval <= v, cand, pos)
            idx = jnp.clip(pos, 1, _NB - 1)
            c0 = plsc.load_gather(ctgtb, [idx - 1])
            c1 = plsc.load_gather(ctgtb, [idx])
            tt = (v - c0) / (c1 - c0 + 1e-6)
            lutv = (idx.astype(jnp.float32) - 1.0 + tt) * (1.0 / 255.0)
            lhalf[pl.ds(i * 16, 16)] = jnp.clip(lutv, 0.0, 1.0)
        pltpu.sync_copy(lhalf, shlut.at[pl.ds(c * _NB + half * 128, 128)])

    plsc.subcore_barrier()

    pltpu.sync_copy(shlut, lutall)
    for h in ah:
        h.wait()
    oh = []
    for c in range(6):
        def _pbody(i, _, c=c):
            sls = [pl.ds(c * _APP + (i * 7 + k) * 16, 16) for k in range(7)]
            vs = [abig[sl] for sl in sls]
            xis = [jnp.clip((v * 255.0).astype(jnp.int32), 0, _NB - 1)
                   + c * _NB for v in vs]
            ys = [plsc.load_gather(lutall, [xi]) for xi in xis]
            for sl, y in zip(sls, ys):
                obig[sl] = jnp.clip(y, 0.0, 1.0)
            return 0
        lax.fori_loop(0, _APP // 16 // 7, _pbody, 0, unroll=2)
        oh.append(pltpu.async_copy(obig.at[pl.ds(c * _APP, _APP)],
                                   out_hbm.at[pl.ds(c * _P + pb, _APP)], osem))
    for h in oh:
        h.wait()


def kernel(source, target):
    N, C, H, W = source.shape
    NC = N * C
    Xs = source.reshape(NC * _P)
    Xt = target.reshape(NC * _P)
    mesh = plsc.VectorSubcoreMesh(
        core_axis_name="c", subcore_axis_name="s",
        num_cores=2, num_subcores=16)
    fn = functools.partial(
        pl.kernel,
        out_type=jax.ShapeDtypeStruct((NC * _P,), jnp.float32),
        mesh=mesh,
        compiler_params=pltpu.CompilerParams(needs_layout_passes=False),
        scratch_types=[
            pltpu.VMEM((_NCH * _XSTR,), jnp.float32),
            pltpu.VMEM((_ROWS * 16,), jnp.float32),
            pltpu.VMEM_SHARED((16, _ROWS * 16), jnp.float32),
            pltpu.VMEM((_SLAB * 16,), jnp.float32),
            pltpu.VMEM((15 * _SLAB * 16,), jnp.float32),
            pltpu.VMEM((_NTAP * 16,), jnp.float32),
            pltpu.VMEM((_NB,), jnp.float32),
            pltpu.VMEM_SHARED((_NB * 16,), jnp.float32),
            pltpu.VMEM((_NB * 16,), jnp.float32),
            pltpu.VMEM_SHARED((_NB * 16,), jnp.float32),
            pltpu.VMEM((_NB,), jnp.float32),
            pltpu.VMEM((128,), jnp.float32),
            pltpu.VMEM_SHARED((6 * _NB,), jnp.float32),
            pltpu.VMEM((6 * _NB,), jnp.float32),
            pltpu.VMEM((6 * _APP,), jnp.float32),
            pltpu.VMEM((6 * _APP,), jnp.float32),
            pltpu.SemaphoreType.DMA,
            pltpu.SemaphoreType.DMA,
            pltpu.SemaphoreType.DMA,
            pltpu.SemaphoreType.DMA,
        ],
    )(_sc_body)
    out = fn(Xs, Xt)
    return out.reshape(N, C, H, W)


# speedup vs baseline: 2.1144x; 1.0098x over previous
import functools
import jax
import jax.numpy as jnp
from jax import lax
from jax.experimental import pallas as pl
from jax.experimental.pallas import tpu as pltpu
from jax.experimental.pallas import tpu_sc as plsc

_NB = 256
_F = 1
_W = 18
_NTAP = 2 * _W + 1
_OFF = _W
_ROWS = 320
_SLAB = 56
_P = 224 * 224
_NCH = 12
_POS = _P // 16
_APP = _P // 32
_SIG_F = 0.01 * 255.0 * _F
_XSTR = _POS + 8


def _sc_body(xs_hbm, xt_hbm, out_hbm, xbuf, cntp, shcnt, slab, tmpbig, gtab,
             histb, shhist, cdfb, ctgtb, lhalf, shlut, lutall,
             abig, obig, dsem, asem, msem, osem):
    core = lax.axis_index("c")
    t = lax.axis_index("s")
    iota16 = lax.broadcasted_iota(jnp.int32, (16,), 0)
    zero16 = jnp.zeros((16,), jnp.float32)
    ones16 = jnp.ones((16,), jnp.float32)
    pb = core * (_P // 2) + t * _APP

    xh = []
    for r in range(_NCH):
        src_ref = xs_hbm if r < 6 else xt_hbm
        off = (r % 6) * _P + t * _POS
        xh.append(pltpu.async_copy(src_ref.at[pl.ds(off, _POS)],
                                   xbuf.at[pl.ds(r * _XSTR, _POS)], dsem))
        xh.append(pltpu.async_copy(src_ref.at[pl.ds(off, 8)],
                                   xbuf.at[pl.ds(r * _XSTR + _POS, 8)], dsem))
    ah = [pltpu.async_copy(xs_hbm.at[pl.ds(c * _P + pb, _APP)],
                           abig.at[pl.ds(c * _APP, _APP)], asem)
          for c in range(6)]

    def _zbody(r, _):
        cntp[pl.ds(r * 16, 16)] = zero16
        return 0
    lax.fori_loop(0, _ROWS, _zbody, 0, unroll=8)

    def _gbody(r, _):
        mf = jnp.full((16,), r, jnp.int32).astype(jnp.float32) - float(_W)
        gtab[pl.ds(r * 16, 16)] = jnp.exp(mf * mf * (-0.5 / (_SIG_F * _SIG_F)))
        return 0
    lax.fori_loop(0, _NTAP, _gbody, 0)
    for h in xh:
        h.wait()

    row_base = jnp.minimum(iota16, _NCH - 1) * _XSTR
    row_lim = row_base + _POS

    def _sbody(p, idxg):
        xs = [plsc.load_gather(xbuf, [idxg + k]) for k in range(8)]
        kis = [jnp.clip((xv * float(255 * _F) + (_OFF + 0.5))
                        .astype(jnp.int32), 0, _ROWS - 1) * 16 + iota16
               for xv in xs]
        for kv in kis:
            plsc.addupdate_scatter(cntp, [kv], ones16)
        nxt = idxg + 8
        return jnp.where(nxt >= row_lim, nxt - _POS, nxt)
    lax.fori_loop(0, _POS // 8, _sbody, row_base + iota16 * 131, unroll=2)

    pltpu.sync_copy(cntp, shcnt.at[t])
    plsc.subcore_barrier()

    r0w = t * (16 * _F * 16)
    mh = [pltpu.async_copy(shcnt.at[0, pl.ds(r0w, _SLAB * 16)], slab, msem)]
    for tab in range(1, 16):
        mh.append(pltpu.async_copy(
            shcnt.at[tab, pl.ds(r0w, _SLAB * 16)],
            tmpbig.at[pl.ds((tab - 1) * _SLAB * 16, _SLAB * 16)], msem))
    for h in mh:
        h.wait()

    def _mbody(r, _):
        sl = pl.ds(r * 16, 16)
        acc = slab[sl]
        accs = [tmpbig[pl.ds(((tab * _SLAB) + r) * 16, 16)]
                for tab in range(15)]
        for a in accs:
            acc = acc + a
        slab[sl] = acc
        return 0
    lax.fori_loop(0, _SLAB, _mbody, 0, unroll=2)

    for i in range(16):
        def _cbody(r, acc, i=i):
            return acc + gtab[pl.ds(r * 16, 16)] * slab[pl.ds((_F * i + r) * 16, 16)]
        histb[pl.ds(i * 16, 16)] = lax.fori_loop(0, _NTAP, _cbody, zero16,
                                                 unroll=4)
    pltpu.sync_copy(histb, shhist.at[pl.ds(t * _NB, _NB)])
    plsc.subcore_barrier()

    @pl.when(t < 12)
    def _lut():
        c = t % 6
        half = t // 6
        pltpu.sync_copy(shhist, cdfb)

        def _abody(j, acc):
            sl = pl.ds(j * 16, 16)
            acc = acc + cdfb[sl]
            cdfb[sl] = acc
            return acc
        total = lax.fori_loop(0, _NB, _abody, zero16)
        s = 1.0 / (total + 1e-6)

        def _nbody(j, _):
            sl = pl.ds(j * 16, 16)
            cdfb[sl] = cdfb[sl] * s
            return 0
        lax.fori_loop(0, _NB, _nbody, 0, unroll=4)
        ctile = jnp.full((16,), c, jnp.int32)
        ttile = ctile + 6
        for kb in range(16):
            kidx = iota16 + kb * 16
            ctgtb[pl.ds(kb * 16, 16)] = plsc.load_gather(
                cdfb, [kidx * 16 + ttile])
        for i in range(8):
            kidx = iota16 + half * 128 + i * 16
            v = plsc.load_gather(cdfb, [kidx * 16 + ctile])
            v = jnp.clip(v, 0.0, 1.0)
            pos = jnp.zeros((16,), jnp.int32)
            for step in (128, 64, 32, 16, 8, 4, 2, 1):
                cand = pos + step
                cval = plsc.load_gather(ctgtb, [cand - 1])
                pos = jnp.where(cval <= v, cand, pos)
            idx = jnp.clip(pos, 1, _NB - 1)
            c0 = plsc.load_gather(ctgtb, [idx - 1])
            c1 = plsc.load_gather(ctgtb, [idx])
            tt = (v - c0) / (c1 - c0 + 1e-6)
            lutv = (idx.astype(jnp.float32) - 1.0 + tt) * (1.0 / 255.0)
            lhalf[pl.ds(i * 16, 16)] = jnp.clip(lutv, 0.0, 1.0)
        pltpu.sync_copy(lhalf, shlut.at[pl.ds(c * _NB + half * 128, 128)])

    plsc.subcore_barrier()

    pltpu.sync_copy(shlut, lutall)
    for h in ah:
        h.wait()
    oh = []
    for c in range(6):
        def _pbody(i, _, c=c):
            sls = [pl.ds(c * _APP + (i * 7 + k) * 16, 16) for k in range(7)]
            vs = [abig[sl] for sl in sls]
            xis = [jnp.clip((v * 255.0).astype(jnp.int32), 0, _NB - 1)
                   + c * _NB for v in vs]
            ys = [plsc.load_gather(lutall, [xi]) for xi in xis]
            for sl, y in zip(sls, ys):
                obig[sl] = jnp.clip(y, 0.0, 1.0)
            return 0
        lax.fori_loop(0, _APP // 16 // 7, _pbody, 0, unroll=2)
        oh.append(pltpu.async_copy(obig.at[pl.ds(c * _APP, _APP)],
                                   out_hbm.at[pl.ds(c * _P + pb, _APP)], osem))
    for h in oh:
        h.wait()


def kernel(source, target):
    N, C, H, W = source.shape
    NC = N * C
    Xs = source.reshape(NC * _P)
    Xt = target.reshape(NC * _P)
    mesh = plsc.VectorSubcoreMesh(
        core_axis_name="c", subcore_axis_name="s",
        num_cores=2, num_subcores=16)
    fn = functools.partial(
        pl.kernel,
        out_type=jax.ShapeDtypeStruct((NC * _P,), jnp.float32),
        mesh=mesh,
        compiler_params=pltpu.CompilerParams(needs_layout_passes=False),
        scratch_types=[
            pltpu.VMEM((_NCH * _XSTR,), jnp.float32),
            pltpu.VMEM((_ROWS * 16,), jnp.float32),
            pltpu.VMEM_SHARED((16, _ROWS * 16), jnp.float32),
            pltpu.VMEM((_SLAB * 16,), jnp.float32),
            pltpu.VMEM((15 * _SLAB * 16,), jnp.float32),
            pltpu.VMEM((_NTAP * 16,), jnp.float32),
            pltpu.VMEM((_NB,), jnp.float32),
            pltpu.VMEM_SHARED((_NB * 16,), jnp.float32),
            pltpu.VMEM((_NB * 16,), jnp.float32),
            pltpu.VMEM((_NB,), jnp.float32),
            pltpu.VMEM((128,), jnp.float32),
            pltpu.VMEM_SHARED((6 * _NB,), jnp.float32),
            pltpu.VMEM((6 * _NB,), jnp.float32),
            pltpu.VMEM((6 * _APP,), jnp.float32),
            pltpu.VMEM((6 * _APP,), jnp.float32),
            pltpu.SemaphoreType.DMA,
            pltpu.SemaphoreType.DMA,
            pltpu.SemaphoreType.DMA,
            pltpu.SemaphoreType.DMA,
        ],
    )(_sc_body)
    out = fn(Xs, Xt)
    return out.reshape(N, C, H, W)
